# Initial kernel scaffold; baseline (speedup 1.0000x reference)
import jax
import jax.numpy as jnp
from jax.experimental import pallas as pl


def kernel(x, gate_W, gate_b, W1, b1, W2, b2):
    raise NotImplementedError("write your pallas kernel here")


# speedup vs baseline: 1.1961x; 1.1961x over previous
import functools

import jax
import jax.numpy as jnp
from jax.experimental import pallas as pl
from jax.experimental.pallas import tpu as pltpu

D = 768
H = 3072
E = 8
S = 2048
EPAD = 128
NH = 4
HB = H // NH
NEG = -1e30


def _gate_kernel(x_ref, gw_ref, gb_ref, w_ref):
    x = x_ref[...]
    scores = jnp.dot(x, gw_ref[...], preferred_element_type=jnp.float32) + gb_ref[...]
    lane = jax.lax.broadcasted_iota(jnp.int32, scores.shape, 1)
    m1 = jnp.max(scores, axis=1, keepdims=True)
    i1 = jnp.min(jnp.where(scores == m1, lane, EPAD), axis=1, keepdims=True)
    masked = jnp.where(lane == i1, NEG, scores)
    m2 = jnp.max(masked, axis=1, keepdims=True)
    i2 = jnp.min(jnp.where(masked == m2, lane, EPAD), axis=1, keepdims=True)
    e2 = jnp.exp(m2 - m1)
    p1 = 1.0 / (1.0 + e2)
    p2 = e2 / (1.0 + e2)
    w_ref[...] = jnp.where(lane == i1, p1, 0.0) + jnp.where(lane == i2, p2, 0.0)


def _expert_kernel(x_ref, w_ref, w1_ref, b1_ref, w2_ref, b2_ref, y_ref):
    e = pl.program_id(0)
    h = pl.program_id(1)
    x = x_ref[...]
    hid = jnp.dot(x, w1_ref[0], preferred_element_type=jnp.float32) + b1_ref[0, 0]
    hid = jnp.maximum(hid, 0.0)
    yblk = jnp.dot(hid, w2_ref[0], preferred_element_type=jnp.float32)
    lane = jax.lax.broadcasted_iota(jnp.int32, w_ref.shape, 1)
    wcol = jnp.sum(jnp.where(lane == e, w_ref[...], 0.0), axis=1, keepdims=True)

    @pl.when(h == 0)
    def _():
        yblk2 = yblk + b2_ref[0, 0]

        @pl.when(e == 0)
        def _():
            y_ref[...] = wcol * yblk2

        @pl.when(e != 0)
        def _():
            y_ref[...] += wcol * yblk2

    @pl.when(h != 0)
    def _():
        y_ref[...] += wcol * yblk


def _moe_dense(x2d, gate_Wp, gate_bp, W1, b1, W2, b2, interpret=False):
    w = pl.pallas_call(
        _gate_kernel,
        out_shape=jax.ShapeDtypeStruct((S, EPAD), jnp.float32),
        interpret=interpret,
    )(x2d, gate_Wp, gate_bp)

    y = pl.pallas_call(
        _expert_kernel,
        grid=(E, NH),
        in_specs=[
            pl.BlockSpec((S, D), lambda e, h: (0, 0)),
            pl.BlockSpec((S, EPAD), lambda e, h: (0, 0)),
            pl.BlockSpec((1, D, HB), lambda e, h: (e, 0, h)),
            pl.BlockSpec((1, 1, HB), lambda e, h: (e, 0, h)),
            pl.BlockSpec((1, HB, D), lambda e, h: (e, h, 0)),
            pl.BlockSpec((1, 1, D), lambda e, h: (e, 0, 0)),
        ],
        out_specs=pl.BlockSpec((S, D), lambda e, h: (0, 0)),
        out_shape=jax.ShapeDtypeStruct((S, D), jnp.float32),
        interpret=interpret,
    )(x2d, w, W1, b1.reshape(E, 1, H), W2, b2.reshape(E, 1, D))
    return y


def kernel(x, gate_W, gate_b, W1, b1, W2, b2):
    x2d = x.reshape(S, D)
    gate_Wp = jnp.pad(gate_W, ((0, 0), (0, EPAD - E)))
    gate_bp = jnp.pad(gate_b.reshape(1, E), ((0, 0), (0, EPAD - E)),
                      constant_values=NEG)
    y = _moe_dense(x2d, gate_Wp, gate_bp, W1, b1, W2, b2)
    return y.reshape(1, S, D)


# speedup vs baseline: 1.6949x; 1.4170x over previous
import functools

import jax
import jax.numpy as jnp
from jax.experimental import pallas as pl
from jax.experimental.pallas import tpu as pltpu
from jax.experimental.pallas import tpu_sc as plsc

D = 768
H = 3072
E = 8
S = 2048
EPAD = 128
NEG = -1e30

BLK = 256
G = 24
P = G * BLK

NC, NS = 2, 16
NW = NC * NS
CHUNK = S // NW

@functools.cache
def _sc_mesh():
    return plsc.VectorSubcoreMesh(core_axis_name="c", subcore_axis_name="s",
                                  num_cores=NC, num_subcores=NS)


def _route_kernel(x_ref, gw_ref, gb_ref, posw_ref, probw_ref, meta_ref):
    x = x_ref[...]
    scores = jnp.dot(x, gw_ref[...], preferred_element_type=jnp.float32) + gb_ref[...]
    lane = jax.lax.broadcasted_iota(jnp.int32, (S, EPAD), 1)
    m1 = jnp.max(scores, axis=1, keepdims=True)
    i1 = jnp.min(jnp.where(scores == m1, lane, EPAD), axis=1, keepdims=True)
    masked = jnp.where(lane == i1, NEG, scores)
    m2 = jnp.max(masked, axis=1, keepdims=True)
    i2 = jnp.min(jnp.where(masked == m2, lane, EPAD), axis=1, keepdims=True)
    e2 = jnp.exp(m2 - m1)
    p1 = 1.0 / (1.0 + e2)
    p2 = e2 / (1.0 + e2)

    oh = (jnp.where(lane == i1, 1.0, 0.0)
          + jnp.where(lane == i2 + E, 1.0, 0.0))
    r_i = jax.lax.broadcasted_iota(jnp.int32, (S, S), 0)
    c_i = jax.lax.broadcasted_iota(jnp.int32, (S, S), 1)
    tril = jnp.where(r_i >= c_i, 1.0, 0.0)
    csum = jnp.dot(tril, oh, preferred_element_type=jnp.float32)
    lane1 = jax.lax.broadcasted_iota(jnp.int32, (1, EPAD), 1)
    tot = csum[S - 1:S, :]
    t0 = jnp.where(lane1 < E, tot, 0.0)
    t1s = jnp.pad(tot[:, E:], ((0, 0), (0, E)))
    cnt = t0 + jnp.where(lane1 < E, t1s, 0.0)

    cnt_i = cnt.astype(jnp.int32)
    pc_i = ((cnt_i + (BLK - 1)) >> 8) << 8
    pc = pc_i.astype(jnp.float32)
    ru = jax.lax.broadcasted_iota(jnp.int32, (EPAD, EPAD), 0)
    cu = jax.lax.broadcasted_iota(jnp.int32, (EPAD, EPAD), 1)
    triu = jnp.where(ru < cu, 1.0, 0.0)
    off = jnp.dot(pc, triu, preferred_element_type=jnp.float32)
    cum = off + pc

    rank0 = jnp.sum(jnp.where(lane == i1, csum, 0.0), axis=1, keepdims=True) - 1.0
    rank1 = jnp.sum(jnp.where(lane == i2 + E, csum, 0.0), axis=1, keepdims=True) - 1.0
    off_b = jnp.broadcast_to(off, (S, EPAD))
    t0_b = jnp.broadcast_to(t0, (S, EPAD))
    off0 = jnp.sum(jnp.where(lane == i1, off_b, 0.0), axis=1, keepdims=True)
    off1 = jnp.sum(jnp.where(lane == i2, off_b, 0.0), axis=1, keepdims=True)
    t0e1 = jnp.sum(jnp.where(lane == i2, t0_b, 0.0), axis=1, keepdims=True)
    pos0 = off0 + rank0
    pos1 = off1 + t0e1 + rank1
    posw_ref[...] = jnp.where(
        lane == 0, pos0, jnp.where(lane == 1, pos1, 0.0)).astype(jnp.int32)
    probw_ref[...] = jnp.where(
        lane < 16, p1, jnp.where(lane < 32, p2, 0.0))

    total_i = jnp.sum(pc_i, axis=1, keepdims=True)
    gclamp = jnp.minimum(lane1 * BLK, total_i - 1)
    cum_i = cum.astype(jnp.int32)
    be = jnp.zeros((1, EPAD), jnp.int32)
    for e in range(E):
        be = be + jnp.where(cum_i[:, e:e + 1] <= gclamp, 1, 0)
    be = jnp.minimum(be, E - 1)
    nact = total_i >> 8
    brow = jnp.minimum(lane1, nact - 1)
    row = jax.lax.broadcasted_iota(jnp.int32, (8, EPAD), 0)
    meta_ref[...] = jnp.where(
        row == 0, jnp.broadcast_to(be, (8, EPAD)),
        jnp.where(row == 1, jnp.broadcast_to(brow, (8, EPAD)),
                  jnp.broadcast_to(nact, (8, EPAD))))


def _scatter_body(x_hbm, pos_hbm, xs_hbm, xv, iv, sem):
    wid = jax.lax.axis_index("s") * NC + jax.lax.axis_index("c")
    base = wid * CHUNK
    pltpu.sync_copy(x_hbm.at[pl.ds(base, CHUNK)], xv)
    for k in range(2):
        pltpu.sync_copy(pos_hbm.at[k, pl.ds(base, CHUNK)], iv)
        pltpu.async_copy(xv, xs_hbm.at[iv], sem).wait()


def _ffn_kernel(be_ref, brow_ref, nact_ref, xs_ref, w1_ref, b1_ref,
                w2_ref, b2_ref, y_ref):
    g = pl.program_id(0)

    @pl.when(g < nact_ref[0])
    def _():
        hid = jnp.dot(xs_ref[...], w1_ref[0],
                      preferred_element_type=jnp.float32) + b1_ref[0, 0]
        hid = jnp.maximum(hid, 0.0)
        y_ref[...] = jnp.dot(hid, w2_ref[0],
                             preferred_element_type=jnp.float32) + b2_ref[0, 0]


def _combine_body(ys_hbm, pos_hbm, prep_hbm, out_hbm, i0v, i1v, y0v, y1v,
                  p0v, p1v, sem):
    wid = jax.lax.axis_index("s") * NC + jax.lax.axis_index("c")
    base = wid * CHUNK
    pltpu.sync_copy(pos_hbm.at[0, pl.ds(base, CHUNK)], i0v)
    pltpu.sync_copy(pos_hbm.at[1, pl.ds(base, CHUNK)], i1v)
    pltpu.sync_copy(prep_hbm.at[0, pl.ds(base, CHUNK)], p0v)
    pltpu.sync_copy(prep_hbm.at[1, pl.ds(base, CHUNK)], p1v)
    cp0 = pltpu.async_copy(ys_hbm.at[i0v], y0v, sem)
    cp1 = pltpu.async_copy(ys_hbm.at[i1v], y1v, sem)
    cp0.wait()
    cp1.wait()

    def row_add(r, carry):
        p0 = p0v[r, :]
        p1 = p1v[r, :]
        for j in range(D // 16):
            sl = pl.ds(j * 16, 16)
            y0v[r, sl] = y0v[r, sl] * p0 + y1v[r, sl] * p1
        return carry

    jax.lax.fori_loop(0, CHUNK, row_add, 0)
    pltpu.sync_copy(y0v, out_hbm.at[pl.ds(base, CHUNK)])


def kernel(x, gate_W, gate_b, W1, b1, W2, b2):
    x2d = x.reshape(S, D)
    gate_Wp = jnp.pad(gate_W, ((0, 0), (0, EPAD - E)))
    gate_bp = jnp.pad(gate_b.reshape(1, E), ((0, 0), (0, EPAD - E)),
                      constant_values=NEG)

    posw, probw, meta = pl.pallas_call(
        _route_kernel,
        out_shape=(
            jax.ShapeDtypeStruct((S, EPAD), jnp.int32),
            jax.ShapeDtypeStruct((S, EPAD), jnp.float32),
            jax.ShapeDtypeStruct((8, EPAD), jnp.int32),
        ),
    )(x2d, gate_Wp, gate_bp)

    pos = jnp.stack([posw[:, 0], posw[:, 1]])
    prep = jnp.stack([probw[:, 0:16], probw[:, 16:32]])
    be = meta[0, :G]
    brow = meta[1, :G]
    nact = meta[2, :1]

    xs = pl.kernel(
        _scatter_body,
        out_type=jax.ShapeDtypeStruct((P, D), jnp.float32),
        mesh=_sc_mesh(),
        scratch_types=[
            pltpu.VMEM((CHUNK, D), jnp.float32),
            pltpu.VMEM((CHUNK,), jnp.int32),
            pltpu.SemaphoreType.DMA,
        ],
    )(x2d, pos)

    grid_spec = pltpu.PrefetchScalarGridSpec(
        num_scalar_prefetch=3,
        grid=(G,),
        in_specs=[
            pl.BlockSpec((BLK, D), lambda g, be, br, na: (br[g], 0)),
            pl.BlockSpec((1, D, H), lambda g, be, br, na: (be[g], 0, 0)),
            pl.BlockSpec((1, 1, H), lambda g, be, br, na: (be[g], 0, 0)),
            pl.BlockSpec((1, H, D), lambda g, be, br, na: (be[g], 0, 0)),
            pl.BlockSpec((1, 1, D), lambda g, be, br, na: (be[g], 0, 0)),
        ],
        out_specs=pl.BlockSpec((BLK, D), lambda g, be, br, na: (br[g], 0)),
    )
    ys = pl.pallas_call(
        _ffn_kernel,
        grid_spec=grid_spec,
        out_shape=jax.ShapeDtypeStruct((P, D), jnp.float32),
    )(be, brow, nact, xs, W1, b1.reshape(E, 1, H), W2,
      b2.reshape(E, 1, D))

    out2d = pl.kernel(
        _combine_body,
        out_type=jax.ShapeDtypeStruct((S, D), jnp.float32),
        mesh=_sc_mesh(),
        scratch_types=[
            pltpu.VMEM((CHUNK,), jnp.int32),
            pltpu.VMEM((CHUNK,), jnp.int32),
            pltpu.VMEM((CHUNK, D), jnp.float32),
            pltpu.VMEM((CHUNK, D), jnp.float32),
            pltpu.VMEM((CHUNK, 16), jnp.float32),
            pltpu.VMEM((CHUNK, 16), jnp.float32),
            pltpu.SemaphoreType.DMA,
        ],
    )(ys, pos, prep)

    return out2d.reshape(1, S, D)


# speedup vs baseline: 1.8594x; 1.0971x over previous
import functools

import jax
import jax.numpy as jnp
from jax.experimental import pallas as pl
from jax.experimental.pallas import tpu as pltpu
from jax.experimental.pallas import tpu_sc as plsc

D = 768
H = 3072
E = 8
S = 2048
EPAD = 128
NEG = -1e30

BLK = 512
G = 16
P = G * BLK

NC, NS = 2, 16
NW = NC * NS
CHUNK = S // NW

@functools.cache
def _sc_mesh():
    return plsc.VectorSubcoreMesh(core_axis_name="c", subcore_axis_name="s",
                                  num_cores=NC, num_subcores=NS)


def _route_kernel(x_ref, gw_ref, gb_ref, posw_ref, probw_ref, meta_ref):
    x = x_ref[...]
    scores = jnp.dot(x, gw_ref[...], preferred_element_type=jnp.float32) + gb_ref[...]
    lane = jax.lax.broadcasted_iota(jnp.int32, (S, EPAD), 1)
    m1 = jnp.max(scores, axis=1, keepdims=True)
    i1 = jnp.min(jnp.where(scores == m1, lane, EPAD), axis=1, keepdims=True)
    masked = jnp.where(lane == i1, NEG, scores)
    m2 = jnp.max(masked, axis=1, keepdims=True)
    i2 = jnp.min(jnp.where(masked == m2, lane, EPAD), axis=1, keepdims=True)
    e2 = jnp.exp(m2 - m1)
    p1 = 1.0 / (1.0 + e2)
    p2 = e2 / (1.0 + e2)

    oh = (jnp.where(lane == i1, 1.0, 0.0)
          + jnp.where(lane == i2 + E, 1.0, 0.0))
    r_i = jax.lax.broadcasted_iota(jnp.int32, (S, S), 0)
    c_i = jax.lax.broadcasted_iota(jnp.int32, (S, S), 1)
    tril = jnp.where(r_i >= c_i, 1.0, 0.0)
    csum = jnp.dot(tril, oh, preferred_element_type=jnp.float32)
    lane1 = jax.lax.broadcasted_iota(jnp.int32, (1, EPAD), 1)
    tot = csum[S - 1:S, :]
    t0 = jnp.where(lane1 < E, tot, 0.0)
    t1s = jnp.pad(tot[:, E:], ((0, 0), (0, E)))
    cnt = t0 + jnp.where(lane1 < E, t1s, 0.0)

    cnt_i = cnt.astype(jnp.int32)
    pc_i = ((cnt_i + (BLK - 1)) >> 9) << 9
    pc = pc_i.astype(jnp.float32)
    ru = jax.lax.broadcasted_iota(jnp.int32, (EPAD, EPAD), 0)
    cu = jax.lax.broadcasted_iota(jnp.int32, (EPAD, EPAD), 1)
    triu = jnp.where(ru < cu, 1.0, 0.0)
    off = jnp.dot(pc, triu, preferred_element_type=jnp.float32)
    cum = off + pc

    rank0 = jnp.sum(jnp.where(lane == i1, csum, 0.0), axis=1, keepdims=True) - 1.0
    rank1 = jnp.sum(jnp.where(lane == i2 + E, csum, 0.0), axis=1, keepdims=True) - 1.0
    off_b = jnp.broadcast_to(off, (S, EPAD))
    t0_b = jnp.broadcast_to(t0, (S, EPAD))
    off0 = jnp.sum(jnp.where(lane == i1, off_b, 0.0), axis=1, keepdims=True)
    off1 = jnp.sum(jnp.where(lane == i2, off_b, 0.0), axis=1, keepdims=True)
    t0e1 = jnp.sum(jnp.where(lane == i2, t0_b, 0.0), axis=1, keepdims=True)
    pos0 = off0 + rank0
    pos1 = off1 + t0e1 + rank1
    posw_ref[...] = jnp.where(
        lane == 0, pos0, jnp.where(lane == 1, pos1, 0.0)).astype(jnp.int32)
    probw_ref[...] = jnp.where(
        lane < 16, p1, jnp.where(lane < 32, p2, 0.0))

    total_i = jnp.sum(pc_i, axis=1, keepdims=True)
    gclamp = jnp.minimum(lane1 * BLK, total_i - 1)
    cum_i = cum.astype(jnp.int32)
    be = jnp.zeros((1, EPAD), jnp.int32)
    for e in range(E):
        be = be + jnp.where(cum_i[:, e:e + 1] <= gclamp, 1, 0)
    be = jnp.minimum(be, E - 1)
    nact = total_i >> 9
    brow = jnp.minimum(lane1, nact - 1)
    row = jax.lax.broadcasted_iota(jnp.int32, (8, EPAD), 0)
    meta_ref[...] = jnp.where(
        row == 0, jnp.broadcast_to(be, (8, EPAD)),
        jnp.where(row == 1, jnp.broadcast_to(brow, (8, EPAD)),
                  jnp.broadcast_to(nact, (8, EPAD))))


def _scatter_body(x_hbm, pos_hbm, xs_hbm, xv, iv, sem):
    wid = jax.lax.axis_index("s") * NC + jax.lax.axis_index("c")
    base = wid * CHUNK
    pltpu.sync_copy(x_hbm.at[pl.ds(base, CHUNK)], xv)
    for k in range(2):
        pltpu.sync_copy(pos_hbm.at[k, pl.ds(base, CHUNK)], iv)
        pltpu.async_copy(xv, xs_hbm.at[iv], sem).wait()


def _ffn_kernel(be_ref, brow_ref, nact_ref, xs_ref, w1_ref, b1_ref,
                w2_ref, b2_ref, y_ref):
    g = pl.program_id(0)

    @pl.when(g < nact_ref[0])
    def _():
        hid = jnp.dot(xs_ref[...], w1_ref[0],
                      preferred_element_type=jnp.float32) + b1_ref[0, 0]
        hid = jnp.maximum(hid, 0.0)
        y_ref[...] = jnp.dot(hid, w2_ref[0],
                             preferred_element_type=jnp.float32) + b2_ref[0, 0]


def _combine_body(ys_hbm, pos_hbm, prep_hbm, out_hbm, i0v, i1v, y0v, y1v,
                  p0v, p1v, sem):
    wid = jax.lax.axis_index("s") * NC + jax.lax.axis_index("c")
    base = wid * CHUNK
    pltpu.sync_copy(pos_hbm.at[0, pl.ds(base, CHUNK)], i0v)
    pltpu.sync_copy(pos_hbm.at[1, pl.ds(base, CHUNK)], i1v)
    pltpu.sync_copy(prep_hbm.at[0, pl.ds(base, CHUNK)], p0v)
    pltpu.sync_copy(prep_hbm.at[1, pl.ds(base, CHUNK)], p1v)
    cp0 = pltpu.async_copy(ys_hbm.at[i0v], y0v, sem)
    cp1 = pltpu.async_copy(ys_hbm.at[i1v], y1v, sem)
    cp0.wait()
    cp1.wait()

    def row_add(r, carry):
        p0 = p0v[r, :]
        p1 = p1v[r, :]
        for j in range(D // 16):
            sl = pl.ds(j * 16, 16)
            y0v[r, sl] = y0v[r, sl] * p0 + y1v[r, sl] * p1
        return carry

    jax.lax.fori_loop(0, CHUNK, row_add, 0)
    pltpu.sync_copy(y0v, out_hbm.at[pl.ds(base, CHUNK)])


def kernel(x, gate_W, gate_b, W1, b1, W2, b2):
    x2d = x.reshape(S, D)
    gate_Wp = jnp.pad(gate_W, ((0, 0), (0, EPAD - E)))
    gate_bp = jnp.pad(gate_b.reshape(1, E), ((0, 0), (0, EPAD - E)),
                      constant_values=NEG)

    posw, probw, meta = pl.pallas_call(
        _route_kernel,
        out_shape=(
            jax.ShapeDtypeStruct((S, EPAD), jnp.int32),
            jax.ShapeDtypeStruct((S, EPAD), jnp.float32),
            jax.ShapeDtypeStruct((8, EPAD), jnp.int32),
        ),
    )(x2d, gate_Wp, gate_bp)

    pos = jnp.stack([posw[:, 0], posw[:, 1]])
    prep = jnp.stack([probw[:, 0:16], probw[:, 16:32]])
    be = meta[0, :G]
    brow = meta[1, :G]
    nact = meta[2, :1]

    xs = pl.kernel(
        _scatter_body,
        out_type=jax.ShapeDtypeStruct((P, D), jnp.float32),
        mesh=_sc_mesh(),
        scratch_types=[
            pltpu.VMEM((CHUNK, D), jnp.float32),
            pltpu.VMEM((CHUNK,), jnp.int32),
            pltpu.SemaphoreType.DMA,
        ],
    )(x2d, pos)

    grid_spec = pltpu.PrefetchScalarGridSpec(
        num_scalar_prefetch=3,
        grid=(G,),
        in_specs=[
            pl.BlockSpec((BLK, D), lambda g, be, br, na: (br[g], 0)),
            pl.BlockSpec((1, D, H), lambda g, be, br, na: (be[g], 0, 0)),
            pl.BlockSpec((1, 1, H), lambda g, be, br, na: (be[g], 0, 0)),
            pl.BlockSpec((1, H, D), lambda g, be, br, na: (be[g], 0, 0)),
            pl.BlockSpec((1, 1, D), lambda g, be, br, na: (be[g], 0, 0)),
        ],
        out_specs=pl.BlockSpec((BLK, D), lambda g, be, br, na: (br[g], 0)),
    )
    ys = pl.pallas_call(
        _ffn_kernel,
        grid_spec=grid_spec,
        out_shape=jax.ShapeDtypeStruct((P, D), jnp.float32),
    )(be, brow, nact, xs, W1, b1.reshape(E, 1, H), W2,
      b2.reshape(E, 1, D))

    out2d = pl.kernel(
        _combine_body,
        out_type=jax.ShapeDtypeStruct((S, D), jnp.float32),
        mesh=_sc_mesh(),
        scratch_types=[
            pltpu.VMEM((CHUNK,), jnp.int32),
            pltpu.VMEM((CHUNK,), jnp.int32),
            pltpu.VMEM((CHUNK, D), jnp.float32),
            pltpu.VMEM((CHUNK, D), jnp.float32),
            pltpu.VMEM((CHUNK, 16), jnp.float32),
            pltpu.VMEM((CHUNK, 16), jnp.float32),
            pltpu.SemaphoreType.DMA,
        ],
    )(ys, pos, prep)

    return out2d.reshape(1, S, D)


def _full_unused():
    pass


# speedup vs baseline: 1.8804x; 1.0113x over previous
import functools

import jax
import jax.numpy as jnp
from jax.experimental import pallas as pl
from jax.experimental.pallas import tpu as pltpu
from jax.experimental.pallas import tpu_sc as plsc

D = 768
H = 3072
E = 8
S = 2048
EPAD = 128
NEG = -1e30

BLK = 512
G = 16
P = G * BLK

NC, NS = 2, 16
NW = NC * NS
CHUNK = S // NW

@functools.cache
def _sc_mesh():
    return plsc.VectorSubcoreMesh(core_axis_name="c", subcore_axis_name="s",
                                  num_cores=NC, num_subcores=NS)


def _route_kernel(x_ref, gw_ref, gb_ref, posw_ref, probw_ref, meta_ref):
    x = x_ref[...]
    scores = jnp.dot(x, gw_ref[...], preferred_element_type=jnp.float32) + gb_ref[...]
    lane = jax.lax.broadcasted_iota(jnp.int32, (S, EPAD), 1)
    m1 = jnp.max(scores, axis=1, keepdims=True)
    i1 = jnp.min(jnp.where(scores == m1, lane, EPAD), axis=1, keepdims=True)
    masked = jnp.where(lane == i1, NEG, scores)
    m2 = jnp.max(masked, axis=1, keepdims=True)
    i2 = jnp.min(jnp.where(masked == m2, lane, EPAD), axis=1, keepdims=True)
    e2 = jnp.exp(m2 - m1)
    p1 = 1.0 / (1.0 + e2)
    p2 = e2 / (1.0 + e2)

    oh = (jnp.where(lane == i1, 1.0, 0.0)
          + jnp.where(lane == i2 + E, 1.0, 0.0))
    r_i = jax.lax.broadcasted_iota(jnp.int32, (S, S), 0)
    c_i = jax.lax.broadcasted_iota(jnp.int32, (S, S), 1)
    tril = jnp.where(r_i >= c_i, 1.0, 0.0)
    csum = jnp.dot(tril, oh, preferred_element_type=jnp.float32)
    lane1 = jax.lax.broadcasted_iota(jnp.int32, (1, EPAD), 1)
    tot = csum[S - 1:S, :]
    t0 = jnp.where(lane1 < E, tot, 0.0)
    t1s = jnp.pad(tot[:, E:], ((0, 0), (0, E)))
    cnt = t0 + jnp.where(lane1 < E, t1s, 0.0)

    cnt_i = cnt.astype(jnp.int32)
    pc_i = ((cnt_i + (BLK - 1)) >> 9) << 9
    pc = pc_i.astype(jnp.float32)
    ru = jax.lax.broadcasted_iota(jnp.int32, (EPAD, EPAD), 0)
    cu = jax.lax.broadcasted_iota(jnp.int32, (EPAD, EPAD), 1)
    triu = jnp.where(ru < cu, 1.0, 0.0)
    off = jnp.dot(pc, triu, preferred_element_type=jnp.float32)
    cum = off + pc

    rank0 = jnp.sum(jnp.where(lane == i1, csum, 0.0), axis=1, keepdims=True) - 1.0
    rank1 = jnp.sum(jnp.where(lane == i2 + E, csum, 0.0), axis=1, keepdims=True) - 1.0
    off_b = jnp.broadcast_to(off, (S, EPAD))
    t0_b = jnp.broadcast_to(t0, (S, EPAD))
    off0 = jnp.sum(jnp.where(lane == i1, off_b, 0.0), axis=1, keepdims=True)
    off1 = jnp.sum(jnp.where(lane == i2, off_b, 0.0), axis=1, keepdims=True)
    t0e1 = jnp.sum(jnp.where(lane == i2, t0_b, 0.0), axis=1, keepdims=True)
    pos0 = off0 + rank0
    pos1 = off1 + t0e1 + rank1
    posw_ref[...] = jnp.where(
        lane == 0, pos0, jnp.where(lane == 1, pos1, 0.0)).astype(jnp.int32)
    probw_ref[...] = jnp.where(
        lane < 16, p1, jnp.where(lane < 32, p2, 0.0))

    total_i = jnp.sum(pc_i, axis=1, keepdims=True)
    gclamp = jnp.minimum(lane1 * BLK, total_i - 1)
    cum_i = cum.astype(jnp.int32)
    be = jnp.zeros((1, EPAD), jnp.int32)
    for e in range(E):
        be = be + jnp.where(cum_i[:, e:e + 1] <= gclamp, 1, 0)
    be = jnp.minimum(be, E - 1)
    nact = total_i >> 9
    brow = jnp.minimum(lane1, nact - 1)
    row = jax.lax.broadcasted_iota(jnp.int32, (8, EPAD), 0)
    meta_ref[...] = jnp.where(
        row == 0, jnp.broadcast_to(be, (8, EPAD)),
        jnp.where(row == 1, jnp.broadcast_to(brow, (8, EPAD)),
                  jnp.broadcast_to(nact, (8, EPAD))))


def _scatter_body(x_hbm, pos_hbm, xs_hbm, xv, iv, sem):
    wid = jax.lax.axis_index("s") * NC + jax.lax.axis_index("c")
    base = wid * CHUNK
    pltpu.sync_copy(x_hbm.at[pl.ds(base, CHUNK)], xv)
    for k in range(2):
        pltpu.sync_copy(pos_hbm.at[k, pl.ds(base, CHUNK)], iv)
        pltpu.async_copy(xv, xs_hbm.at[iv], sem).wait()


def _ffn_kernel(be_ref, brow_ref, nact_ref, xs_ref, w1_ref, b1_ref,
                w2_ref, b2_ref, y_ref):
    g = pl.program_id(0)

    @pl.when(g < nact_ref[0])
    def _():
        hid = jnp.dot(xs_ref[...], w1_ref[0],
                      preferred_element_type=jnp.float32) + b1_ref[0, 0]
        hid = jnp.maximum(hid, 0.0)
        y_ref[...] = jnp.dot(hid, w2_ref[0],
                             preferred_element_type=jnp.float32) + b2_ref[0, 0]


def _combine_body(ys_hbm, pos_hbm, probw_hbm, out_hbm, i0v, i1v, y0v, y1v,
                  qwv, sem):
    wid = jax.lax.axis_index("s") * NC + jax.lax.axis_index("c")
    base = wid * CHUNK
    pltpu.sync_copy(pos_hbm.at[0, pl.ds(base, CHUNK)], i0v)
    pltpu.sync_copy(pos_hbm.at[1, pl.ds(base, CHUNK)], i1v)
    pltpu.sync_copy(probw_hbm.at[pl.ds(base * EPAD, CHUNK * EPAD)], qwv)
    cp0 = pltpu.async_copy(ys_hbm.at[i0v], y0v, sem)
    cp1 = pltpu.async_copy(ys_hbm.at[i1v], y1v, sem)
    cp0.wait()
    cp1.wait()

    def row_add(r, carry):
        p0 = qwv[pl.ds(r * EPAD, 16)]
        p1 = qwv[pl.ds(r * EPAD + 16, 16)]
        for j in range(D // 16):
            sl = pl.ds(j * 16, 16)
            y0v[r, sl] = y0v[r, sl] * p0 + y1v[r, sl] * p1
        return carry

    jax.lax.fori_loop(0, CHUNK, row_add, 0)
    pltpu.sync_copy(y0v, out_hbm.at[pl.ds(base, CHUNK)])


def kernel(x, gate_W, gate_b, W1, b1, W2, b2):
    x2d = x.reshape(S, D)
    gate_Wp = jnp.pad(gate_W, ((0, 0), (0, EPAD - E)))
    gate_bp = jnp.pad(gate_b.reshape(1, E), ((0, 0), (0, EPAD - E)),
                      constant_values=NEG)

    posw, probw, meta = pl.pallas_call(
        _route_kernel,
        out_shape=(
            jax.ShapeDtypeStruct((S, EPAD), jnp.int32),
            jax.ShapeDtypeStruct((S, EPAD), jnp.float32),
            jax.ShapeDtypeStruct((8, EPAD), jnp.int32),
        ),
    )(x2d, gate_Wp, gate_bp)

    pos = jnp.stack([posw[:, 0], posw[:, 1]])
    be = meta[0, :G]
    brow = meta[1, :G]
    nact = meta[2, :1]

    xs = pl.kernel(
        _scatter_body,
        out_type=jax.ShapeDtypeStruct((P, D), jnp.float32),
        mesh=_sc_mesh(),
        scratch_types=[
            pltpu.VMEM((CHUNK, D), jnp.float32),
            pltpu.VMEM((CHUNK,), jnp.int32),
            pltpu.SemaphoreType.DMA,
        ],
    )(x2d, pos)

    grid_spec = pltpu.PrefetchScalarGridSpec(
        num_scalar_prefetch=3,
        grid=(G,),
        in_specs=[
            pl.BlockSpec((BLK, D), lambda g, be, br, na: (br[g], 0)),
            pl.BlockSpec((1, D, H), lambda g, be, br, na: (be[g], 0, 0)),
            pl.BlockSpec((1, 1, H), lambda g, be, br, na: (be[g], 0, 0)),
            pl.BlockSpec((1, H, D), lambda g, be, br, na: (be[g], 0, 0)),
            pl.BlockSpec((1, 1, D), lambda g, be, br, na: (be[g], 0, 0)),
        ],
        out_specs=pl.BlockSpec((BLK, D), lambda g, be, br, na: (br[g], 0)),
    )
    ys = pl.pallas_call(
        _ffn_kernel,
        grid_spec=grid_spec,
        out_shape=jax.ShapeDtypeStruct((P, D), jnp.float32),
    )(be, brow, nact, xs, W1, b1.reshape(E, 1, H), W2,
      b2.reshape(E, 1, D))

    out2d = pl.kernel(
        _combine_body,
        out_type=jax.ShapeDtypeStruct((S, D), jnp.float32),
        mesh=_sc_mesh(),
        scratch_types=[
            pltpu.VMEM((CHUNK,), jnp.int32),
            pltpu.VMEM((CHUNK,), jnp.int32),
            pltpu.VMEM((CHUNK, D), jnp.float32),
            pltpu.VMEM((CHUNK, D), jnp.float32),
            pltpu.VMEM((CHUNK * EPAD,), jnp.float32),
            pltpu.SemaphoreType.DMA,
        ],
    )(ys, pos, probw.reshape(S * EPAD))

    return out2d.reshape(1, S, D)


def _full_unused():
    pass


# speedup vs baseline: 1.8906x; 1.0054x over previous
import functools

import jax
import jax.numpy as jnp
from jax.experimental import pallas as pl
from jax.experimental.pallas import tpu as pltpu
from jax.experimental.pallas import tpu_sc as plsc

D = 768
H = 3072
E = 8
S = 2048
EPAD = 128
NEG = -1e30

BLK = 512
G = 16
P = G * BLK

NC, NS = 2, 16
NW = NC * NS
CHUNK = S // NW

@functools.cache
def _sc_mesh():
    return plsc.VectorSubcoreMesh(core_axis_name="c", subcore_axis_name="s",
                                  num_cores=NC, num_subcores=NS)


def _route_kernel(x_ref, gw_ref, gb_ref, posw_ref, probw_ref, meta_ref):
    x = x_ref[...]
    scores = jnp.dot(x, gw_ref[...], preferred_element_type=jnp.float32) + gb_ref[...]
    lane = jax.lax.broadcasted_iota(jnp.int32, (S, EPAD), 1)
    m1 = jnp.max(scores, axis=1, keepdims=True)
    i1 = jnp.min(jnp.where(scores == m1, lane, EPAD), axis=1, keepdims=True)
    masked = jnp.where(lane == i1, NEG, scores)
    m2 = jnp.max(masked, axis=1, keepdims=True)
    i2 = jnp.min(jnp.where(masked == m2, lane, EPAD), axis=1, keepdims=True)
    e2 = jnp.exp(m2 - m1)
    p1 = 1.0 / (1.0 + e2)
    p2 = e2 / (1.0 + e2)

    oh = (jnp.where(lane == i1, 1.0, 0.0)
          + jnp.where(lane == i2 + E, 1.0, 0.0))
    r_i = jax.lax.broadcasted_iota(jnp.int32, (S, S), 0)
    c_i = jax.lax.broadcasted_iota(jnp.int32, (S, S), 1)
    tril = jnp.where(r_i >= c_i, 1.0, 0.0)
    csum = jnp.dot(tril, oh, preferred_element_type=jnp.float32)
    lane1 = jax.lax.broadcasted_iota(jnp.int32, (1, EPAD), 1)
    tot = csum[S - 1:S, :]
    t0 = jnp.where(lane1 < E, tot, 0.0)
    t1s = jnp.pad(tot[:, E:], ((0, 0), (0, E)))
    cnt = t0 + jnp.where(lane1 < E, t1s, 0.0)

    cnt_i = cnt.astype(jnp.int32)
    pc_i = ((cnt_i + (BLK - 1)) >> 9) << 9
    pc = pc_i.astype(jnp.float32)
    ru = jax.lax.broadcasted_iota(jnp.int32, (EPAD, EPAD), 0)
    cu = jax.lax.broadcasted_iota(jnp.int32, (EPAD, EPAD), 1)
    triu = jnp.where(ru < cu, 1.0, 0.0)
    off = jnp.dot(pc, triu, preferred_element_type=jnp.float32)
    cum = off + pc

    rank0 = jnp.sum(jnp.where(lane == i1, csum, 0.0), axis=1, keepdims=True) - 1.0
    rank1 = jnp.sum(jnp.where(lane == i2 + E, csum, 0.0), axis=1, keepdims=True) - 1.0
    off_b = jnp.broadcast_to(off, (S, EPAD))
    t0_b = jnp.broadcast_to(t0, (S, EPAD))
    off0 = jnp.sum(jnp.where(lane == i1, off_b, 0.0), axis=1, keepdims=True)
    off1 = jnp.sum(jnp.where(lane == i2, off_b, 0.0), axis=1, keepdims=True)
    t0e1 = jnp.sum(jnp.where(lane == i2, t0_b, 0.0), axis=1, keepdims=True)
    pos0 = off0 + rank0
    pos1 = off1 + t0e1 + rank1
    posw_ref[...] = jnp.where(
        lane == 0, pos0, jnp.where(lane == 1, pos1, 0.0)).astype(jnp.int32)
    probw_ref[...] = jnp.where(
        lane < 16, p1, jnp.where(lane < 32, p2, 0.0))

    total_i = jnp.sum(pc_i, axis=1, keepdims=True)
    gclamp = jnp.minimum(lane1 * BLK, total_i - 1)
    cum_i = cum.astype(jnp.int32)
    be = jnp.zeros((1, EPAD), jnp.int32)
    for e in range(E):
        be = be + jnp.where(cum_i[:, e:e + 1] <= gclamp, 1, 0)
    be = jnp.minimum(be, E - 1)
    nact = total_i >> 9
    brow = jnp.minimum(lane1, nact - 1)
    row = jax.lax.broadcasted_iota(jnp.int32, (8, EPAD), 0)
    meta_ref[...] = jnp.where(
        row == 0, jnp.broadcast_to(be, (8, EPAD)),
        jnp.where(row == 1, jnp.broadcast_to(brow, (8, EPAD)),
                  jnp.broadcast_to(nact, (8, EPAD))))


def _scatter_body(x_hbm, pos_hbm, xs_hbm, xv, iv, sem):
    wid = jax.lax.axis_index("s") * NC + jax.lax.axis_index("c")
    base = wid * CHUNK
    pltpu.sync_copy(x_hbm.at[pl.ds(base, CHUNK)], xv)
    for k in range(2):
        pltpu.sync_copy(pos_hbm.at[k, pl.ds(base, CHUNK)], iv)
        pltpu.async_copy(xv, xs_hbm.at[iv], sem).wait()


def _ffn_kernel(be_ref, brow_ref, nact_ref, xs_ref, w1_ref, b1_ref,
                w2_ref, b2_ref, y_ref):
    g = pl.program_id(0)

    @pl.when(g < nact_ref[0])
    def _():
        hid = jnp.dot(xs_ref[...].astype(jnp.bfloat16),
                      w1_ref[0].astype(jnp.bfloat16),
                      preferred_element_type=jnp.float32) + b1_ref[0, 0]
        hid = jnp.maximum(hid, 0.0)
        y_ref[...] = jnp.dot(hid.astype(jnp.bfloat16),
                             w2_ref[0].astype(jnp.bfloat16),
                             preferred_element_type=jnp.float32) + b2_ref[0, 0]


def _combine_body(ys_hbm, pos_hbm, probw_hbm, out_hbm, i0v, i1v, y0v, y1v,
                  qwv, sem):
    wid = jax.lax.axis_index("s") * NC + jax.lax.axis_index("c")
    base = wid * CHUNK
    pltpu.sync_copy(pos_hbm.at[0, pl.ds(base, CHUNK)], i0v)
    pltpu.sync_copy(pos_hbm.at[1, pl.ds(base, CHUNK)], i1v)
    pltpu.sync_copy(probw_hbm.at[pl.ds(base * EPAD, CHUNK * EPAD)], qwv)
    cp0 = pltpu.async_copy(ys_hbm.at[i0v], y0v, sem)
    cp1 = pltpu.async_copy(ys_hbm.at[i1v], y1v, sem)
    cp0.wait()
    cp1.wait()

    def row_add(r, carry):
        p0 = qwv[pl.ds(r * EPAD, 16)]
        p1 = qwv[pl.ds(r * EPAD + 16, 16)]
        for j in range(D // 16):
            sl = pl.ds(j * 16, 16)
            y0v[r, sl] = y0v[r, sl] * p0 + y1v[r, sl] * p1
        return carry

    jax.lax.fori_loop(0, CHUNK, row_add, 0)
    pltpu.sync_copy(y0v, out_hbm.at[pl.ds(base, CHUNK)])


def kernel(x, gate_W, gate_b, W1, b1, W2, b2):
    x2d = x.reshape(S, D)
    gate_Wp = jnp.pad(gate_W, ((0, 0), (0, EPAD - E)))
    gate_bp = jnp.pad(gate_b.reshape(1, E), ((0, 0), (0, EPAD - E)),
                      constant_values=NEG)

    posw, probw, meta = pl.pallas_call(
        _route_kernel,
        out_shape=(
            jax.ShapeDtypeStruct((S, EPAD), jnp.int32),
            jax.ShapeDtypeStruct((S, EPAD), jnp.float32),
            jax.ShapeDtypeStruct((8, EPAD), jnp.int32),
        ),
    )(x2d, gate_Wp, gate_bp)

    pos = jnp.stack([posw[:, 0], posw[:, 1]])
    be = meta[0, :G]
    brow = meta[1, :G]
    nact = meta[2, :1]

    xs = pl.kernel(
        _scatter_body,
        out_type=jax.ShapeDtypeStruct((P, D), jnp.float32),
        mesh=_sc_mesh(),
        scratch_types=[
            pltpu.VMEM((CHUNK, D), jnp.float32),
            pltpu.VMEM((CHUNK,), jnp.int32),
            pltpu.SemaphoreType.DMA,
        ],
    )(x2d, pos)

    grid_spec = pltpu.PrefetchScalarGridSpec(
        num_scalar_prefetch=3,
        grid=(G,),
        in_specs=[
            pl.BlockSpec((BLK, D), lambda g, be, br, na: (br[g], 0)),
            pl.BlockSpec((1, D, H), lambda g, be, br, na: (be[g], 0, 0)),
            pl.BlockSpec((1, 1, H), lambda g, be, br, na: (be[g], 0, 0)),
            pl.BlockSpec((1, H, D), lambda g, be, br, na: (be[g], 0, 0)),
            pl.BlockSpec((1, 1, D), lambda g, be, br, na: (be[g], 0, 0)),
        ],
        out_specs=pl.BlockSpec((BLK, D), lambda g, be, br, na: (br[g], 0)),
    )
    ys = pl.pallas_call(
        _ffn_kernel,
        grid_spec=grid_spec,
        out_shape=jax.ShapeDtypeStruct((P, D), jnp.float32),
    )(be, brow, nact, xs, W1, b1.reshape(E, 1, H), W2,
      b2.reshape(E, 1, D))

    out2d = pl.kernel(
        _combine_body,
        out_type=jax.ShapeDtypeStruct((S, D), jnp.float32),
        mesh=_sc_mesh(),
        scratch_types=[
            pltpu.VMEM((CHUNK,), jnp.int32),
            pltpu.VMEM((CHUNK,), jnp.int32),
            pltpu.VMEM((CHUNK, D), jnp.float32),
            pltpu.VMEM((CHUNK, D), jnp.float32),
            pltpu.VMEM((CHUNK * EPAD,), jnp.float32),
            pltpu.SemaphoreType.DMA,
        ],
    )(ys, pos, probw.reshape(S * EPAD))

    return out2d.reshape(1, S, D)


def _full_unused():
    pass
